# baseline (speedup 1.0000x reference)
import jax
import jax.numpy as jnp
from jax import lax
from jax.experimental import pallas as pl
from jax.experimental.pallas import tpu as pltpu
from jax.experimental.pallas import tpu_sc as plsc

_ALPHA = 0.5
_N = 4096
_D = 1024
_L = 16
_CHUNKS = _N // _L


def _ksom_body(xt_hbm, w_hbm, wout_hbm, wins_hbm, xt_v, w_v, wins_v):
    c = lax.axis_index("c")
    s = lax.axis_index("s")
    wid = s * 2 + c

    @pl.when(wid == 0)
    def _():
        pltpu.sync_copy(xt_hbm, xt_v)
        pltpu.sync_copy(w_hbm, w_v)

        row0 = w_v[0, pl.ds(0, _L)]
        row1 = w_v[1, pl.ds(0, _L)]
        init = (row0[0], row1[0], row0[1], row1[1])

        lane = lax.iota(jnp.int32, _L)

        def chunk(k, carry):
            base = k * (2 * _L)
            for half in range(2):
                off = base + half * _L
                a_vec = xt_v[0, pl.ds(off, _L)]
                b_vec = xt_v[1, pl.ds(off, _L)]
                pack = jnp.int32(0)
                for j in range(_L):
                    w00, w10, w01, w11 = carry
                    a = a_vec[j]
                    b = b_vec[j]
                    e1 = a - w00
                    e2 = a - w10
                    d1 = e1 * e1
                    d2 = e2 * e2
                    win0 = d1 < d2
                    pack = pack | ((d2 <= d1).astype(jnp.int32) << j)
                    n00 = w00 + _ALPHA * e1
                    n10 = w10 + _ALPHA * e2
                    n01 = w01 + _ALPHA * (b - w01)
                    n11 = w11 + _ALPHA * (b - w11)
                    carry = (
                        jnp.where(win0, n00, w00),
                        jnp.where(win0, w10, n10),
                        jnp.where(win0, n01, w01),
                        jnp.where(win0, w11, n11),
                    )
                win_vec = (jnp.broadcast_to(pack, (_L,)) >> lane) & 1
                wins_v[pl.ds(off, _L)] = win_vec
            return carry

        w00, w10, w01, w11 = lax.fori_loop(0, _CHUNKS // 2, chunk, init)

        new0 = jnp.where(lane == 0, w00, jnp.where(lane == 1, w01, row0))
        new1 = jnp.where(lane == 0, w10, jnp.where(lane == 1, w11, row1))
        w_v[0, pl.ds(0, _L)] = new0
        w_v[1, pl.ds(0, _L)] = new1

        pltpu.sync_copy(w_v, wout_hbm)
        pltpu.sync_copy(wins_v, wins_hbm)


@jax.jit
def kernel(x, weights):
    xt = lax.slice(x, (0, 0), (_N, 2)).T
    mesh = plsc.VectorSubcoreMesh(core_axis_name="c", subcore_axis_name="s")
    run = pl.kernel(
        _ksom_body,
        out_type=(
            jax.ShapeDtypeStruct((2, _D), jnp.float32),
            jax.ShapeDtypeStruct((_N,), jnp.int32),
        ),
        mesh=mesh,
        scratch_types=(
            pltpu.VMEM((2, _N), jnp.float32),
            pltpu.VMEM((2, _D), jnp.float32),
            pltpu.VMEM((_N,), jnp.int32),
        ),
    )
    final_w, wins = run(xt, weights)
    return final_w, wins


# speedup vs baseline: 1.0017x; 1.0017x over previous
import jax
import jax.numpy as jnp
from jax import lax
from jax.experimental import pallas as pl
from jax.experimental.pallas import tpu as pltpu
from jax.experimental.pallas import tpu_sc as plsc

_ALPHA = 0.5
_N = 4096
_D = 1024
_L = 16
_CHUNKS = _N // _L


def _ksom_body(xf_hbm, w_hbm, wout_hbm, wins_hbm, xf_v, w_v, wins_v, sem1, sem2):
    c = lax.axis_index("c")
    s = lax.axis_index("s")
    wid = s * 2 + c

    @pl.when(wid == 0)
    def _():
        cp1 = pltpu.async_copy(xf_hbm, xf_v, sem1)
        cp2 = pltpu.async_copy(w_hbm, w_v, sem2)
        cp1.wait()
        cp2.wait()

        row0 = w_v[0, pl.ds(0, _L)]
        row1 = w_v[1, pl.ds(0, _L)]
        init = (row0[0], row1[0], row0[1], row1[1])

        lane = lax.iota(jnp.int32, _L)

        def chunk(k, carry):
            base = k * _L
            a_vec = xf_v[pl.ds(base, _L)]
            b_vec = xf_v[pl.ds(_N + base, _L)]
            pack = jnp.int32(0)
            for j in range(_L):
                w00, w10, w01, w11 = carry
                a = a_vec[j]
                b = b_vec[j]
                e1 = a - w00
                e2 = a - w10
                d1 = e1 * e1
                d2 = e2 * e2
                win0 = d1 < d2
                pack = pack | (jnp.where(win0, 0, 1) << j)
                n00 = w00 + _ALPHA * e1
                n10 = w10 + _ALPHA * e2
                n01 = w01 + _ALPHA * (b - w01)
                n11 = w11 + _ALPHA * (b - w11)
                carry = (
                    jnp.where(win0, n00, w00),
                    jnp.where(win0, w10, n10),
                    jnp.where(win0, n01, w01),
                    jnp.where(win0, w11, n11),
                )
            win_vec = (jnp.broadcast_to(pack, (_L,)) >> lane) & 1
            wins_v[pl.ds(base, _L)] = win_vec
            return carry

        w00, w10, w01, w11 = lax.fori_loop(0, _CHUNKS, chunk, init)

        new0 = jnp.where(lane == 0, w00, jnp.where(lane == 1, w01, row0))
        new1 = jnp.where(lane == 0, w10, jnp.where(lane == 1, w11, row1))
        w_v[0, pl.ds(0, _L)] = new0
        w_v[1, pl.ds(0, _L)] = new1

        cp3 = pltpu.async_copy(w_v, wout_hbm, sem1)
        cp4 = pltpu.async_copy(wins_v, wins_hbm, sem2)
        cp3.wait()
        cp4.wait()


@jax.jit
def kernel(x, weights):
    xf = jnp.concatenate(
        [lax.slice(x, (0, 0), (_N, 1)), lax.slice(x, (0, 1), (_N, 2))], axis=0
    ).reshape(2 * _N)
    mesh = plsc.VectorSubcoreMesh(core_axis_name="c", subcore_axis_name="s")
    run = pl.kernel(
        _ksom_body,
        out_type=(
            jax.ShapeDtypeStruct((2, _D), jnp.float32),
            jax.ShapeDtypeStruct((_N,), jnp.int32),
        ),
        mesh=mesh,
        scratch_types=(
            pltpu.VMEM((2 * _N,), jnp.float32),
            pltpu.VMEM((2, _D), jnp.float32),
            pltpu.VMEM((_N,), jnp.int32),
            pltpu.SemaphoreType.DMA,
            pltpu.SemaphoreType.DMA,
        ),
    )
    final_w, wins = run(xf, weights)
    return final_w, wins


# speedup vs baseline: 1.0104x; 1.0088x over previous
import jax
import jax.numpy as jnp
from jax import lax
from jax.experimental import pallas as pl
from jax.experimental.pallas import tpu as pltpu
from jax.experimental.pallas import tpu_sc as plsc

_ALPHA = 0.5
_N = 4096
_D = 1024
_L = 16
_CHUNKS = _N // _L


def _ksom_body(xt_hbm, w_hbm, wout_hbm, wins_hbm, xt_v, w_v, wins_v):
    c = lax.axis_index("c")
    s = lax.axis_index("s")
    wid = s * 2 + c

    @pl.when(wid == 0)
    def _():
        pltpu.sync_copy(xt_hbm, xt_v)
        pltpu.sync_copy(w_hbm, w_v)

        row0 = w_v[0, pl.ds(0, _L)]
        row1 = w_v[1, pl.ds(0, _L)]
        init = (row0[0], row1[0], row0[1], row1[1])

        lane = lax.iota(jnp.int32, _L)

        def chunk(k, carry):
            base = k * _L
            a_vec = xt_v[0, pl.ds(base, _L)]
            b_vec = xt_v[1, pl.ds(base, _L)]
            pack = jnp.int32(0)
            for j in range(_L):
                w00, w10, w01, w11 = carry
                a = a_vec[j]
                b = b_vec[j]
                e1 = a - w00
                e2 = a - w10
                d1 = e1 * e1
                d2 = e2 * e2
                win0 = d1 < d2
                pack = pack | (jnp.where(win0, 0, 1) << j)
                n00 = w00 + _ALPHA * e1
                n10 = w10 + _ALPHA * e2
                n01 = w01 + _ALPHA * (b - w01)
                n11 = w11 + _ALPHA * (b - w11)
                carry = (
                    jnp.where(win0, n00, w00),
                    jnp.where(win0, w10, n10),
                    jnp.where(win0, n01, w01),
                    jnp.where(win0, w11, n11),
                )
            win_vec = (jnp.broadcast_to(pack, (_L,)) >> lane) & 1
            wins_v[pl.ds(base, _L)] = win_vec
            return carry

        w00, w10, w01, w11 = lax.fori_loop(0, _CHUNKS, chunk, init)

        new0 = jnp.where(lane == 0, w00, jnp.where(lane == 1, w01, row0))
        new1 = jnp.where(lane == 0, w10, jnp.where(lane == 1, w11, row1))
        w_v[0, pl.ds(0, _L)] = new0
        w_v[1, pl.ds(0, _L)] = new1

        pltpu.sync_copy(w_v, wout_hbm)
        pltpu.sync_copy(wins_v, wins_hbm)


@jax.jit
def kernel(x, weights):
    xt = lax.slice(x, (0, 0), (_N, 2)).T
    mesh = plsc.VectorSubcoreMesh(core_axis_name="c", subcore_axis_name="s")
    run = pl.kernel(
        _ksom_body,
        out_type=(
            jax.ShapeDtypeStruct((2, _D), jnp.float32),
            jax.ShapeDtypeStruct((_N,), jnp.int32),
        ),
        mesh=mesh,
        scratch_types=(
            pltpu.VMEM((2, _N), jnp.float32),
            pltpu.VMEM((2, _D), jnp.float32),
            pltpu.VMEM((_N,), jnp.int32),
        ),
    )
    final_w, wins = run(xt, weights)
    return final_w, wins
